# baseline (speedup 1.0000x reference)
import functools

import jax
import jax.numpy as jnp
from jax import lax
from jax.experimental import pallas as pl
from jax.experimental.pallas import tpu as pltpu
from jax.experimental.pallas import tpu_sc as plsc

_NC = 2
_NS = 16
_NW = _NC * _NS
_LANES = 16
_CH = 128


def _flat_worker_id():
    return lax.axis_index("c") * _NS + lax.axis_index("s")


@functools.partial(jax.jit, static_argnames=("n", "e", "rb"))
def _degree_partials(e2, *, n, e, rb):
    nchunk_tot = e // _CH
    nchunk = nchunk_tot // _NW
    nextra = nchunk_tot - nchunk * _NW
    nb = n // rb
    mesh = plsc.VectorSubcoreMesh(core_axis_name="c", subcore_axis_name="s")

    @functools.partial(
        pl.kernel,
        out_type=jax.ShapeDtypeStruct((nb, _NW, rb), jnp.float32),
        mesh=mesh,
        scratch_types=[
            pltpu.VMEM((nchunk, _CH), jnp.int32),
            pltpu.VMEM((1, _CH), jnp.int32),
            pltpu.VMEM((n,), jnp.float32),
        ],
        compiler_params=pltpu.CompilerParams(
            needs_layout_passes=False, use_tc_tiling_on_sc=False),
    )
    def deg_kernel(e2_hbm, out_hbm, idx_v, ex_v, acc_v):
        wid = _flat_worker_id()
        pltpu.sync_copy(e2_hbm.at[1, pl.ds(wid * nchunk, nchunk)], idx_v)

        @pl.when(wid < nextra)
        def _():
            pltpu.sync_copy(e2_hbm.at[1, pl.ds(nchunk * _NW + wid, 1)], ex_v)

        zeros = jnp.zeros((_LANES,), jnp.float32)

        def zero_body(i, _):
            acc_v[pl.ds(i * _LANES, _LANES)] = zeros
            return 0

        lax.fori_loop(0, n // _LANES, zero_body, 0)
        ones = jnp.ones((_LANES,), jnp.float32)

        def hist_body(i, _):
            for j in range(_CH // _LANES):
                idx = idx_v[i, pl.ds(j * _LANES, _LANES)]
                plsc.addupdate_scatter(acc_v, [idx], ones)
            return 0

        lax.fori_loop(0, nchunk, hist_body, 0)

        @pl.when(wid < nextra)
        def _():
            for j in range(_CH // _LANES):
                idx = ex_v[0, pl.ds(j * _LANES, _LANES)]
                plsc.addupdate_scatter(acc_v, [idx], ones)

        for blk in range(nb):
            pltpu.sync_copy(acc_v.at[pl.ds(blk * rb, rb)],
                            out_hbm.at[blk, wid])

    return deg_kernel(e2)


@functools.partial(jax.jit, static_argnames=("n", "f", "e"))
def _edge_aggregate(y, e2, *, n, f, e):
    nchunk_tot = e // _CH
    nchunk = nchunk_tot // _NW
    nextra = nchunk_tot - nchunk * _NW
    rpt = n // _NS
    mesh = plsc.VectorSubcoreMesh(core_axis_name="c", subcore_axis_name="s")

    @functools.partial(
        pl.kernel,
        out_type=jax.ShapeDtypeStruct((n, 128), jnp.float32),
        mesh=mesh,
        scratch_types=[
            pltpu.VMEM((nchunk, _CH), jnp.int32),
            pltpu.VMEM((nchunk, _CH), jnp.int32),
            pltpu.VMEM((1, _CH), jnp.int32),
            pltpu.VMEM((1, _CH), jnp.int32),
            pltpu.VMEM((_CH, f), jnp.float32),
            pltpu.VMEM((_CH, f), jnp.float32),
            pltpu.VMEM_SHARED((n, f), jnp.float32),
            pltpu.SemaphoreType.DMA,
            pltpu.SemaphoreType.DMA,
        ],
        compiler_params=pltpu.CompilerParams(use_tc_tiling_on_sc=False),
    )
    def agg_kernel(y_hbm, e2_hbm, out_hbm, src_v, dst_v, exs_v, exd_v,
                   rows0_v, rows1_v, acc_sh, sem0, sem1):
        cid = lax.axis_index("c")
        sid = lax.axis_index("s")
        wid = cid * _NS + sid
        pltpu.sync_copy(e2_hbm.at[0, pl.ds(wid * nchunk, nchunk)], src_v)
        pltpu.sync_copy(e2_hbm.at[1, pl.ds(wid * nchunk, nchunk)], dst_v)

        @pl.when(wid < nextra)
        def _():
            pltpu.sync_copy(e2_hbm.at[0, pl.ds(nchunk * _NW + wid, 1)], exs_v)
            pltpu.sync_copy(e2_hbm.at[1, pl.ds(nchunk * _NW + wid, 1)], exd_v)

        zeros = jnp.zeros((_LANES,), jnp.float32)

        def zero_body(i, _):
            for j in range(f // _LANES):
                rows0_v[i, pl.ds(j * _LANES, _LANES)] = zeros
            return 0

        lax.fori_loop(0, _CH, zero_body, 0)
        nfull, tail = rpt // _CH, rpt % _CH
        for k in range(nfull):
            pltpu.sync_copy(rows0_v,
                            acc_sh.at[pl.ds(sid * rpt + k * _CH, _CH)])
        if tail:
            pltpu.sync_copy(rows0_v.at[pl.ds(0, tail)],
                            acc_sh.at[pl.ds(sid * rpt + nfull * _CH, tail)])
        plsc.subcore_barrier()

        pltpu.async_copy(y_hbm.at[src_v.at[0]], rows0_v, sem0)

        def edge_body(jj, _):
            j0 = jj * 2
            pltpu.async_copy(y_hbm.at[src_v.at[j0 + 1]], rows1_v, sem1)
            pltpu.make_async_copy(y_hbm.at[src_v.at[j0]], rows0_v,
                                  sem0).wait()
            pltpu.sync_copy(rows0_v, acc_sh.at[dst_v.at[j0]], add=True)

            @pl.when(jj < nchunk // 2 - 1)
            def _():
                pltpu.async_copy(y_hbm.at[src_v.at[j0 + 2]], rows0_v, sem0)

            pltpu.make_async_copy(y_hbm.at[src_v.at[j0 + 1]], rows1_v,
                                  sem1).wait()
            pltpu.sync_copy(rows1_v, acc_sh.at[dst_v.at[j0 + 1]], add=True)
            return 0

        lax.fori_loop(0, nchunk // 2, edge_body, 0)

        @pl.when(wid < nextra)
        def _():
            pltpu.async_copy(y_hbm.at[exs_v.at[0]], rows0_v, sem0).wait()
            pltpu.sync_copy(rows0_v, acc_sh.at[exd_v.at[0]], add=True)

        plsc.subcore_barrier()
        pltpu.sync_copy(acc_sh.at[pl.ds(sid * rpt, rpt)],
                        out_hbm.at[pl.ds(sid * rpt, rpt), pl.ds(cid * f, f)])

    return agg_kernel(y, e2)


def _dis_from(degp_ref):
    deg = jnp.sum(degp_ref[0], axis=0) + 1.0
    return lax.rsqrt(deg)[:, None]


@functools.partial(jax.jit, static_argnames=("rb",))
def _tc_stage0(x, W1, ri2, W2b, *, rb):
    n, d = x.shape
    h = W1.shape[1]
    b = ri2.shape[0]
    nb = n // rb

    def body(x_ref, w1_ref, ri_ref, w2b_ref, xw_ref, r2_ref, rv_acc):
        i = pl.program_id(0)
        xb = x_ref[...]
        xw_ref[...] = jnp.dot(xb, w1_ref[...],
                              preferred_element_type=jnp.float32)
        gidx = i * rb + lax.broadcasted_iota(jnp.int32, (b, rb), 1)
        rsel = (ri_ref[...] == gidx).astype(jnp.float32)

        @pl.when(i == 0)
        def _():
            rv_acc[...] = jnp.zeros_like(rv_acc)

        rv_acc[...] += jnp.dot(rsel, xb, preferred_element_type=jnp.float32)

        @pl.when(i == nb - 1)
        def _():
            r2_ref[...] = jnp.dot(jnp.maximum(rv_acc[...], 0.0),
                                  w2b_ref[...],
                                  preferred_element_type=jnp.float32)

    return pl.pallas_call(
        body,
        grid=(nb,),
        in_specs=[
            pl.BlockSpec((rb, d), lambda i: (i, 0)),
            pl.BlockSpec((d, h), lambda i: (0, 0)),
            pl.BlockSpec((b, 1), lambda i: (0, 0)),
            pl.BlockSpec((d, h), lambda i: (0, 0)),
        ],
        out_specs=[
            pl.BlockSpec((rb, h), lambda i: (i, 0)),
            pl.BlockSpec((b, h), lambda i: (0, 0)),
        ],
        out_shape=[
            jax.ShapeDtypeStruct((n, h), jnp.float32),
            jax.ShapeDtypeStruct((b, h), jnp.float32),
        ],
        scratch_shapes=[pltpu.VMEM((b, d), jnp.float32)],
    )(x, W1, ri2, W2b)


@functools.partial(jax.jit, static_argnames=("rb",))
def _tc_stage1(degp, xw, *, rb):
    n, h = xw.shape
    nb = n // rb

    def body(degp_ref, xw_ref, y1_ref):
        dis = _dis_from(degp_ref)
        y1_ref[...] = dis * xw_ref[...]

    return pl.pallas_call(
        body,
        grid=(nb,),
        in_specs=[
            pl.BlockSpec((1, _NW, rb), lambda i: (i, 0, 0)),
            pl.BlockSpec((rb, h), lambda i: (i, 0)),
        ],
        out_specs=pl.BlockSpec((rb, h), lambda i: (i, 0)),
        out_shape=jax.ShapeDtypeStruct((n, h), jnp.float32),
    )(degp, xw)


@functools.partial(jax.jit, static_argnames=("rb",))
def _tc_stage2(S1, degp, y1, W2t, R2, bt3, b1r, *, rb):
    n = S1.shape[0]
    h = y1.shape[1]
    o = W2t.shape[1]
    b = R2.shape[0]
    nb = n // rb

    def body(s1_ref, degp_ref, y1_ref, w2t_ref, r2_ref, bt_ref, b1_ref,
             y2_ref, x2_ref):
        dis = _dis_from(degp_ref)
        sp = s1_ref[...]
        x2 = dis * (sp[:, :h] + sp[:, h:] + y1_ref[...]) + b1_ref[...]
        x2_ref[...] = x2
        r = jnp.maximum(x2, 0.0)
        bt = bt_ref[0, 0, :]
        onehot = (bt[:, None]
                  == lax.broadcasted_iota(jnp.int32, (rb, b), 1)
                  ).astype(jnp.float32)
        hw2 = (jnp.dot(r, w2t_ref[...], preferred_element_type=jnp.float32)
               + jnp.dot(onehot, r2_ref[...],
                         preferred_element_type=jnp.float32))
        y2_ref[...] = dis * hw2

    return pl.pallas_call(
        body,
        grid=(nb,),
        in_specs=[
            pl.BlockSpec((rb, 128), lambda i: (i, 0)),
            pl.BlockSpec((1, _NW, rb), lambda i: (i, 0, 0)),
            pl.BlockSpec((rb, h), lambda i: (i, 0)),
            pl.BlockSpec((h, o), lambda i: (0, 0)),
            pl.BlockSpec((b, o), lambda i: (0, 0)),
            pl.BlockSpec((1, 1, rb), lambda i: (i, 0, 0)),
            pl.BlockSpec((1, h), lambda i: (0, 0)),
        ],
        out_specs=[
            pl.BlockSpec((rb, o), lambda i: (i, 0)),
            pl.BlockSpec((rb, h), lambda i: (i, 0)),
        ],
        out_shape=[
            jax.ShapeDtypeStruct((n, o), jnp.float32),
            jax.ShapeDtypeStruct((n, h), jnp.float32),
        ],
    )(S1, degp, y1, W2t, R2, bt3, b1r)


@functools.partial(jax.jit, static_argnames=("rb",))
def _tc_stage3a(x2, bt3, ri2, *, rb):
    n, h = x2.shape
    b = ri2.shape[0]
    nb = n // rb

    def body(x2_ref, bt_ref, ri_ref, x2r_ref, cnt_ref):
        i = pl.program_id(0)
        bt = bt_ref[0, 0, :]
        onehot_t = (lax.broadcasted_iota(jnp.int32, (b, rb), 0)
                    == bt[None, :]).astype(jnp.float32)
        gidx = i * rb + lax.broadcasted_iota(jnp.int32, (b, rb), 1)
        rsel = (ri_ref[...] == gidx).astype(jnp.float32)

        @pl.when(i == 0)
        def _():
            x2r_ref[...] = jnp.zeros_like(x2r_ref)
            cnt_ref[...] = jnp.zeros_like(cnt_ref)

        x2r_ref[...] += jnp.dot(rsel, x2_ref[...],
                                preferred_element_type=jnp.float32)
        cnt_ref[...] += jnp.sum(onehot_t, axis=1, keepdims=True)

    return pl.pallas_call(
        body,
        grid=(nb,),
        in_specs=[
            pl.BlockSpec((rb, h), lambda i: (i, 0)),
            pl.BlockSpec((1, 1, rb), lambda i: (i, 0, 0)),
            pl.BlockSpec((b, 1), lambda i: (0, 0)),
        ],
        out_specs=[
            pl.BlockSpec((b, h), lambda i: (0, 0)),
            pl.BlockSpec((b, 1), lambda i: (0, 0)),
        ],
        out_shape=[
            jax.ShapeDtypeStruct((b, h), jnp.float32),
            jax.ShapeDtypeStruct((b, 1), jnp.float32),
        ],
    )(x2, bt3, ri2)


@functools.partial(jax.jit, static_argnames=("rb",))
def _tc_stage3b(S2, degp, y2, bt3, x2r, cnt, b2r, *, rb):
    n = S2.shape[0]
    o = y2.shape[1]
    b = x2r.shape[0]
    h = x2r.shape[1]
    nb = n // rb

    def body(s2_ref, degp_ref, y2_ref, bt_ref, x2r_ref, cnt_ref, b2_ref,
             out_ref, sums_acc):
        i = pl.program_id(0)
        dis = _dis_from(degp_ref)
        sp = s2_ref[...]
        g = jnp.maximum(
            dis * (sp[:, :o] + sp[:, o:] + y2_ref[...]) + b2_ref[...], 0.0)
        bt = bt_ref[0, 0, :]
        onehot_t = (lax.broadcasted_iota(jnp.int32, (b, rb), 0)
                    == bt[None, :]).astype(jnp.float32)

        @pl.when(i == 0)
        def _():
            sums_acc[...] = jnp.zeros_like(sums_acc)

        sums_acc[...] += jnp.dot(onehot_t, g,
                                 preferred_element_type=jnp.float32)

        @pl.when(i == nb - 1)
        def _():
            c = cnt_ref[...]
            mean = sums_acc[...] / jnp.maximum(c, 1.0)
            right = x2r_ref[...] * (c > 0.0).astype(jnp.float32)
            out_ref[...] = jnp.concatenate([mean, right], axis=1)

    return pl.pallas_call(
        body,
        grid=(nb,),
        in_specs=[
            pl.BlockSpec((rb, 128), lambda i: (i, 0)),
            pl.BlockSpec((1, _NW, rb), lambda i: (i, 0, 0)),
            pl.BlockSpec((rb, o), lambda i: (i, 0)),
            pl.BlockSpec((1, 1, rb), lambda i: (i, 0, 0)),
            pl.BlockSpec((b, h), lambda i: (0, 0)),
            pl.BlockSpec((b, 1), lambda i: (0, 0)),
            pl.BlockSpec((1, o), lambda i: (0, 0)),
        ],
        out_specs=pl.BlockSpec((b, o + h), lambda i: (0, 0)),
        out_shape=jax.ShapeDtypeStruct((b, o + h), jnp.float32),
        scratch_shapes=[pltpu.VMEM((b, o), jnp.float32)],
    )(S2, degp, y2, bt3, x2r, cnt, b2r)


def kernel(x, edge_index, root_index, batch, W1, b1, W2, b2):
    n, d = x.shape
    e = edge_index.shape[1]
    h = W1.shape[1]
    o = W2.shape[1]
    b = root_index.shape[0]
    rb = 1000

    e2 = edge_index.astype(jnp.int32).reshape(2, e // _CH, _CH)
    bt3 = batch.astype(jnp.int32).reshape(n // rb, 1, rb)
    ri2 = root_index.astype(jnp.int32).reshape(b, 1)
    b1r = b1.reshape(1, h)
    b2r = b2.reshape(1, o)
    W2t = W2[:h]
    W2b = W2[h:]

    degp = _degree_partials(e2, n=n, e=e, rb=rb)
    xw, R2 = _tc_stage0(x, W1, ri2, W2b, rb=rb)
    y1 = _tc_stage1(degp, xw, rb=rb)
    S1 = _edge_aggregate(y1, e2, n=n, f=h, e=e)
    y2, x2 = _tc_stage2(S1, degp, y1, W2t, R2, bt3, b1r, rb=rb)
    S2 = _edge_aggregate(y2, e2, n=n, f=o, e=e)
    x2r, cnt = _tc_stage3a(x2, bt3, ri2, rb=rb)
    return _tc_stage3b(S2, degp, y2, bt3, x2r, cnt, b2r, rb=rb)


# speedup vs baseline: 1.0418x; 1.0418x over previous
import functools

import jax
import jax.numpy as jnp
from jax import lax
from jax.experimental import pallas as pl
from jax.experimental.pallas import tpu as pltpu
from jax.experimental.pallas import tpu_sc as plsc

_NC = 2
_NS = 16
_NW = _NC * _NS
_LANES = 16
_CH = 128


def _flat_worker_id():
    return lax.axis_index("c") * _NS + lax.axis_index("s")


@functools.partial(jax.jit, static_argnames=("n", "e", "rb"))
def _degree_partials(e2, *, n, e, rb):
    nchunk_tot = e // _CH
    nchunk = nchunk_tot // _NW
    nextra = nchunk_tot - nchunk * _NW
    nb = n // rb
    mesh = plsc.VectorSubcoreMesh(core_axis_name="c", subcore_axis_name="s")

    @functools.partial(
        pl.kernel,
        out_type=jax.ShapeDtypeStruct((nb, _NW, rb), jnp.float32),
        mesh=mesh,
        scratch_types=[
            pltpu.VMEM((nchunk, _CH), jnp.int32),
            pltpu.VMEM((1, _CH), jnp.int32),
            pltpu.VMEM((n,), jnp.float32),
        ],
        compiler_params=pltpu.CompilerParams(
            needs_layout_passes=False, use_tc_tiling_on_sc=False),
    )
    def deg_kernel(e2_hbm, out_hbm, idx_v, ex_v, acc_v):
        wid = _flat_worker_id()
        pltpu.sync_copy(e2_hbm.at[1, pl.ds(wid * nchunk, nchunk)], idx_v)

        @pl.when(wid < nextra)
        def _():
            pltpu.sync_copy(e2_hbm.at[1, pl.ds(nchunk * _NW + wid, 1)], ex_v)

        zeros = jnp.zeros((_LANES,), jnp.float32)

        def zero_body(i, _):
            acc_v[pl.ds(i * _LANES, _LANES)] = zeros
            return 0

        lax.fori_loop(0, n // _LANES, zero_body, 0)
        ones = jnp.ones((_LANES,), jnp.float32)

        def hist_body(i, _):
            for j in range(_CH // _LANES):
                idx = idx_v[i, pl.ds(j * _LANES, _LANES)]
                plsc.addupdate_scatter(acc_v, [idx], ones)
            return 0

        lax.fori_loop(0, nchunk, hist_body, 0)

        @pl.when(wid < nextra)
        def _():
            for j in range(_CH // _LANES):
                idx = ex_v[0, pl.ds(j * _LANES, _LANES)]
                plsc.addupdate_scatter(acc_v, [idx], ones)

        for blk in range(nb):
            pltpu.sync_copy(acc_v.at[pl.ds(blk * rb, rb)],
                            out_hbm.at[blk, wid])

    return deg_kernel(e2)


@functools.partial(jax.jit, static_argnames=("n", "f", "e"))
def _edge_aggregate(y, e2, *, n, f, e):
    nchunk_tot = e // _CH
    nchunk = nchunk_tot // _NW
    nextra = nchunk_tot - nchunk * _NW
    rpt = n // _NS
    mesh = plsc.VectorSubcoreMesh(core_axis_name="c", subcore_axis_name="s")

    @functools.partial(
        pl.kernel,
        out_type=jax.ShapeDtypeStruct((n, 128), jnp.float32),
        mesh=mesh,
        scratch_types=[
            pltpu.VMEM((nchunk, _CH), jnp.int32),
            pltpu.VMEM((nchunk, _CH), jnp.int32),
            pltpu.VMEM((1, _CH), jnp.int32),
            pltpu.VMEM((1, _CH), jnp.int32),
            pltpu.VMEM((_CH, f), jnp.float32),
            pltpu.VMEM((_CH, f), jnp.float32),
            pltpu.VMEM_SHARED((n, f), jnp.float32),
            pltpu.SemaphoreType.DMA,
            pltpu.SemaphoreType.DMA,
        ],
        compiler_params=pltpu.CompilerParams(use_tc_tiling_on_sc=False),
    )
    def agg_kernel(y_hbm, e2_hbm, out_hbm, src_v, dst_v, exs_v, exd_v,
                   rows0_v, rows1_v, acc_sh, sem0, sem1):
        cid = lax.axis_index("c")
        sid = lax.axis_index("s")
        wid = cid * _NS + sid
        pltpu.sync_copy(e2_hbm.at[0, pl.ds(wid * nchunk, nchunk)], src_v)
        pltpu.sync_copy(e2_hbm.at[1, pl.ds(wid * nchunk, nchunk)], dst_v)

        @pl.when(wid < nextra)
        def _():
            pltpu.sync_copy(e2_hbm.at[0, pl.ds(nchunk * _NW + wid, 1)], exs_v)
            pltpu.sync_copy(e2_hbm.at[1, pl.ds(nchunk * _NW + wid, 1)], exd_v)

        zeros = jnp.zeros((_LANES,), jnp.float32)

        def zero_body(i, _):
            for j in range(f // _LANES):
                rows0_v[i, pl.ds(j * _LANES, _LANES)] = zeros
            return 0

        lax.fori_loop(0, _CH, zero_body, 0)
        nfull, tail = rpt // _CH, rpt % _CH
        for k in range(nfull):
            pltpu.sync_copy(rows0_v,
                            acc_sh.at[pl.ds(sid * rpt + k * _CH, _CH)])
        if tail:
            pltpu.sync_copy(rows0_v.at[pl.ds(0, tail)],
                            acc_sh.at[pl.ds(sid * rpt + nfull * _CH, tail)])
        plsc.subcore_barrier()

        pltpu.async_copy(y_hbm.at[src_v.at[0]], rows0_v, sem0)

        def edge_body(jj, _):
            j0 = jj * 2
            pltpu.async_copy(y_hbm.at[src_v.at[j0 + 1]], rows1_v, sem1)
            pltpu.make_async_copy(y_hbm.at[src_v.at[j0]], rows0_v,
                                  sem0).wait()
            pltpu.sync_copy(rows0_v, acc_sh.at[dst_v.at[j0]], add=True)

            @pl.when(jj < nchunk // 2 - 1)
            def _():
                pltpu.async_copy(y_hbm.at[src_v.at[j0 + 2]], rows0_v, sem0)

            pltpu.make_async_copy(y_hbm.at[src_v.at[j0 + 1]], rows1_v,
                                  sem1).wait()
            pltpu.sync_copy(rows1_v, acc_sh.at[dst_v.at[j0 + 1]], add=True)
            return 0

        lax.fori_loop(0, nchunk // 2, edge_body, 0)

        @pl.when(wid < nextra)
        def _():
            pltpu.async_copy(y_hbm.at[exs_v.at[0]], rows0_v, sem0).wait()
            pltpu.sync_copy(rows0_v, acc_sh.at[exd_v.at[0]], add=True)

        plsc.subcore_barrier()
        pltpu.sync_copy(acc_sh.at[pl.ds(sid * rpt, rpt)],
                        out_hbm.at[pl.ds(sid * rpt, rpt), pl.ds(cid * f, f)])

    return agg_kernel(y, e2)


def _dis_from(degp_ref):
    deg = jnp.sum(degp_ref[0], axis=0) + 1.0
    return lax.rsqrt(deg)[:, None]


@functools.partial(jax.jit, static_argnames=("rb",))
def _tc_stage0(x, W1, ri2, W2b, *, rb):
    n, d = x.shape
    h = W1.shape[1]
    b = ri2.shape[0]
    nb = n // rb

    def body(x_ref, w1_ref, ri_ref, w2b_ref, xw_ref, r2_ref, rv_acc):
        i = pl.program_id(0)
        xb = x_ref[...]
        xw_ref[...] = jnp.dot(xb, w1_ref[...],
                              preferred_element_type=jnp.float32)
        gidx = i * rb + lax.broadcasted_iota(jnp.int32, (b, rb), 1)
        rsel = (ri_ref[...] == gidx).astype(jnp.float32)

        @pl.when(i == 0)
        def _():
            rv_acc[...] = jnp.zeros_like(rv_acc)

        rv_acc[...] += jnp.dot(rsel, xb, preferred_element_type=jnp.float32)

        @pl.when(i == nb - 1)
        def _():
            r2_ref[...] = jnp.dot(jnp.maximum(rv_acc[...], 0.0),
                                  w2b_ref[...],
                                  preferred_element_type=jnp.float32)

    return pl.pallas_call(
        body,
        grid=(nb,),
        in_specs=[
            pl.BlockSpec((rb, d), lambda i: (i, 0)),
            pl.BlockSpec((d, h), lambda i: (0, 0)),
            pl.BlockSpec((b, 1), lambda i: (0, 0)),
            pl.BlockSpec((d, h), lambda i: (0, 0)),
        ],
        out_specs=[
            pl.BlockSpec((rb, h), lambda i: (i, 0)),
            pl.BlockSpec((b, h), lambda i: (0, 0)),
        ],
        out_shape=[
            jax.ShapeDtypeStruct((n, h), jnp.float32),
            jax.ShapeDtypeStruct((b, h), jnp.float32),
        ],
        scratch_shapes=[pltpu.VMEM((b, d), jnp.float32)],
    )(x, W1, ri2, W2b)


@functools.partial(jax.jit, static_argnames=("rb",))
def _tc_stage1(degp, xw, *, rb):
    n, h = xw.shape
    nb = n // rb

    def body(degp_ref, xw_ref, y1_ref):
        dis = _dis_from(degp_ref)
        y1_ref[...] = dis * xw_ref[...]

    return pl.pallas_call(
        body,
        grid=(nb,),
        in_specs=[
            pl.BlockSpec((1, _NW, rb), lambda i: (i, 0, 0)),
            pl.BlockSpec((rb, h), lambda i: (i, 0)),
        ],
        out_specs=pl.BlockSpec((rb, h), lambda i: (i, 0)),
        out_shape=jax.ShapeDtypeStruct((n, h), jnp.float32),
    )(degp, xw)


@functools.partial(jax.jit, static_argnames=("rb",))
def _tc_stage2(S1, degp, y1, W2t, R2, bt3, b1r, *, rb):
    n = S1.shape[0]
    h = y1.shape[1]
    o = W2t.shape[1]
    b = R2.shape[0]
    nb = n // rb

    def body(s1_ref, degp_ref, y1_ref, w2t_ref, r2_ref, bt_ref, b1_ref,
             y2_ref, x2_ref):
        dis = _dis_from(degp_ref)
        sp = s1_ref[...]
        x2 = dis * (sp[:, :h] + sp[:, h:] + y1_ref[...]) + b1_ref[...]
        x2_ref[...] = x2
        r = jnp.maximum(x2, 0.0)
        bt = bt_ref[0, 0, :]
        onehot = (bt[:, None]
                  == lax.broadcasted_iota(jnp.int32, (rb, b), 1)
                  ).astype(jnp.float32)
        hw2 = (jnp.dot(r, w2t_ref[...], preferred_element_type=jnp.float32)
               + jnp.dot(onehot, r2_ref[...],
                         preferred_element_type=jnp.float32))
        y2_ref[...] = dis * hw2

    return pl.pallas_call(
        body,
        grid=(nb,),
        in_specs=[
            pl.BlockSpec((rb, 128), lambda i: (i, 0)),
            pl.BlockSpec((1, _NW, rb), lambda i: (i, 0, 0)),
            pl.BlockSpec((rb, h), lambda i: (i, 0)),
            pl.BlockSpec((h, o), lambda i: (0, 0)),
            pl.BlockSpec((b, o), lambda i: (0, 0)),
            pl.BlockSpec((1, 1, rb), lambda i: (i, 0, 0)),
            pl.BlockSpec((1, h), lambda i: (0, 0)),
        ],
        out_specs=[
            pl.BlockSpec((rb, o), lambda i: (i, 0)),
            pl.BlockSpec((rb, h), lambda i: (i, 0)),
        ],
        out_shape=[
            jax.ShapeDtypeStruct((n, o), jnp.float32),
            jax.ShapeDtypeStruct((n, h), jnp.float32),
        ],
    )(S1, degp, y1, W2t, R2, bt3, b1r)


@functools.partial(jax.jit, static_argnames=("rb",))
def _tc_stage3a(x2, bt3, ri2, *, rb):
    n, h = x2.shape
    b = ri2.shape[0]
    nb = n // rb

    def body(x2_ref, bt_ref, ri_ref, x2r_ref, cnt_ref):
        i = pl.program_id(0)
        bt = bt_ref[0, 0, :]
        onehot_t = (lax.broadcasted_iota(jnp.int32, (b, rb), 0)
                    == bt[None, :]).astype(jnp.float32)
        gidx = i * rb + lax.broadcasted_iota(jnp.int32, (b, rb), 1)
        rsel = (ri_ref[...] == gidx).astype(jnp.float32)

        @pl.when(i == 0)
        def _():
            x2r_ref[...] = jnp.zeros_like(x2r_ref)
            cnt_ref[...] = jnp.zeros_like(cnt_ref)

        x2r_ref[...] += jnp.dot(rsel, x2_ref[...],
                                preferred_element_type=jnp.float32)
        cnt_ref[...] += jnp.sum(onehot_t, axis=1, keepdims=True)

    return pl.pallas_call(
        body,
        grid=(nb,),
        in_specs=[
            pl.BlockSpec((rb, h), lambda i: (i, 0)),
            pl.BlockSpec((1, 1, rb), lambda i: (i, 0, 0)),
            pl.BlockSpec((b, 1), lambda i: (0, 0)),
        ],
        out_specs=[
            pl.BlockSpec((b, h), lambda i: (0, 0)),
            pl.BlockSpec((b, 1), lambda i: (0, 0)),
        ],
        out_shape=[
            jax.ShapeDtypeStruct((b, h), jnp.float32),
            jax.ShapeDtypeStruct((b, 1), jnp.float32),
        ],
    )(x2, bt3, ri2)


@functools.partial(jax.jit, static_argnames=("rb",))
def _tc_stage3b(S2, degp, y2, bt3, x2r, cnt, b2r, *, rb):
    n = S2.shape[0]
    o = y2.shape[1]
    b = x2r.shape[0]
    h = x2r.shape[1]
    nb = n // rb

    def body(s2_ref, degp_ref, y2_ref, bt_ref, x2r_ref, cnt_ref, b2_ref,
             out_ref, sums_acc):
        i = pl.program_id(0)
        dis = _dis_from(degp_ref)
        sp = s2_ref[...]
        g = jnp.maximum(
            dis * (sp[:, :o] + sp[:, o:] + y2_ref[...]) + b2_ref[...], 0.0)
        bt = bt_ref[0, 0, :]
        onehot_t = (lax.broadcasted_iota(jnp.int32, (b, rb), 0)
                    == bt[None, :]).astype(jnp.float32)

        @pl.when(i == 0)
        def _():
            sums_acc[...] = jnp.zeros_like(sums_acc)

        sums_acc[...] += jnp.dot(onehot_t, g,
                                 preferred_element_type=jnp.float32)

        @pl.when(i == nb - 1)
        def _():
            c = cnt_ref[...]
            mean = sums_acc[...] / jnp.maximum(c, 1.0)
            right = x2r_ref[...] * (c > 0.0).astype(jnp.float32)
            out_ref[...] = jnp.concatenate([mean, right], axis=1)

    return pl.pallas_call(
        body,
        grid=(nb,),
        in_specs=[
            pl.BlockSpec((rb, 128), lambda i: (i, 0)),
            pl.BlockSpec((1, _NW, rb), lambda i: (i, 0, 0)),
            pl.BlockSpec((rb, o), lambda i: (i, 0)),
            pl.BlockSpec((1, 1, rb), lambda i: (i, 0, 0)),
            pl.BlockSpec((b, h), lambda i: (0, 0)),
            pl.BlockSpec((b, 1), lambda i: (0, 0)),
            pl.BlockSpec((1, o), lambda i: (0, 0)),
        ],
        out_specs=pl.BlockSpec((b, o + h), lambda i: (0, 0)),
        out_shape=jax.ShapeDtypeStruct((b, o + h), jnp.float32),
        scratch_shapes=[pltpu.VMEM((b, o), jnp.float32)],
    )(S2, degp, y2, bt3, x2r, cnt, b2r)


def kernel(x, edge_index, root_index, batch, W1, b1, W2, b2):
    n, d = x.shape
    e = edge_index.shape[1]
    h = W1.shape[1]
    o = W2.shape[1]
    b = root_index.shape[0]
    rb = 2000

    e2 = edge_index.astype(jnp.int32).reshape(2, e // _CH, _CH)
    bt3 = batch.astype(jnp.int32).reshape(n // rb, 1, rb)
    ri2 = root_index.astype(jnp.int32).reshape(b, 1)
    b1r = b1.reshape(1, h)
    b2r = b2.reshape(1, o)
    W2t = W2[:h]
    W2b = W2[h:]

    degp = _degree_partials(e2, n=n, e=e, rb=rb)
    xw, R2 = _tc_stage0(x, W1, ri2, W2b, rb=rb)
    y1 = _tc_stage1(degp, xw, rb=rb)
    S1 = _edge_aggregate(y1, e2, n=n, f=h, e=e)
    y2, x2 = _tc_stage2(S1, degp, y1, W2t, R2, bt3, b1r, rb=rb)
    S2 = _edge_aggregate(y2, e2, n=n, f=o, e=e)
    x2r, cnt = _tc_stage3a(x2, bt3, ri2, rb=rb)
    return _tc_stage3b(S2, degp, y2, bt3, x2r, cnt, b2r, rb=rb)


# speedup vs baseline: 1.0465x; 1.0045x over previous
import functools

import jax
import jax.numpy as jnp
from jax import lax
from jax.experimental import pallas as pl
from jax.experimental.pallas import tpu as pltpu
from jax.experimental.pallas import tpu_sc as plsc

_NC = 2
_NS = 16
_NW = _NC * _NS
_LANES = 16
_CH = 128


def _flat_worker_id():
    return lax.axis_index("c") * _NS + lax.axis_index("s")


@functools.partial(jax.jit, static_argnames=("n", "e", "rb"))
def _degree_partials(e2, *, n, e, rb):
    nchunk_tot = e // _CH
    nchunk = nchunk_tot // _NW
    nextra = nchunk_tot - nchunk * _NW
    nb = n // rb
    mesh = plsc.VectorSubcoreMesh(core_axis_name="c", subcore_axis_name="s")

    @functools.partial(
        pl.kernel,
        out_type=jax.ShapeDtypeStruct((nb, _NW, rb), jnp.float32),
        mesh=mesh,
        scratch_types=[
            pltpu.VMEM((nchunk, _CH), jnp.int32),
            pltpu.VMEM((1, _CH), jnp.int32),
            pltpu.VMEM((n,), jnp.float32),
        ],
        compiler_params=pltpu.CompilerParams(
            needs_layout_passes=False, use_tc_tiling_on_sc=False),
    )
    def deg_kernel(e2_hbm, out_hbm, idx_v, ex_v, acc_v):
        wid = _flat_worker_id()
        pltpu.sync_copy(e2_hbm.at[1, pl.ds(wid * nchunk, nchunk)], idx_v)

        @pl.when(wid < nextra)
        def _():
            pltpu.sync_copy(e2_hbm.at[1, pl.ds(nchunk * _NW + wid, 1)], ex_v)

        zeros = jnp.zeros((_LANES,), jnp.float32)

        def zero_body(i, _):
            acc_v[pl.ds(i * _LANES, _LANES)] = zeros
            return 0

        lax.fori_loop(0, n // _LANES, zero_body, 0)
        ones = jnp.ones((_LANES,), jnp.float32)

        def hist_body(i, _):
            for j in range(_CH // _LANES):
                idx = idx_v[i, pl.ds(j * _LANES, _LANES)]
                plsc.addupdate_scatter(acc_v, [idx], ones)
            return 0

        lax.fori_loop(0, nchunk, hist_body, 0)

        @pl.when(wid < nextra)
        def _():
            for j in range(_CH // _LANES):
                idx = ex_v[0, pl.ds(j * _LANES, _LANES)]
                plsc.addupdate_scatter(acc_v, [idx], ones)

        for blk in range(nb):
            pltpu.sync_copy(acc_v.at[pl.ds(blk * rb, rb)],
                            out_hbm.at[blk, wid])

    return deg_kernel(e2)


@functools.partial(jax.jit, static_argnames=("n", "f", "e"))
def _edge_aggregate(y, e2, *, n, f, e):
    nchunk_tot = e // _CH
    nchunk = nchunk_tot // _NW
    nextra = nchunk_tot - nchunk * _NW
    rpt = n // _NS
    mesh = plsc.VectorSubcoreMesh(core_axis_name="c", subcore_axis_name="s")

    @functools.partial(
        pl.kernel,
        out_type=jax.ShapeDtypeStruct((n, 128), jnp.float32),
        mesh=mesh,
        scratch_types=[
            pltpu.VMEM((nchunk, _CH), jnp.int32),
            pltpu.VMEM((nchunk, _CH), jnp.int32),
            pltpu.VMEM((1, _CH), jnp.int32),
            pltpu.VMEM((1, _CH), jnp.int32),
            pltpu.VMEM((_CH, f), jnp.float32),
            pltpu.VMEM((_CH, f), jnp.float32),
            pltpu.VMEM_SHARED((n, f), jnp.float32),
            pltpu.SemaphoreType.DMA,
            pltpu.SemaphoreType.DMA,
        ],
        compiler_params=pltpu.CompilerParams(use_tc_tiling_on_sc=False),
    )
    def agg_kernel(y_hbm, e2_hbm, out_hbm, src_v, dst_v, exs_v, exd_v,
                   rows0_v, rows1_v, acc_sh, sem0, sem1):
        cid = lax.axis_index("c")
        sid = lax.axis_index("s")
        wid = cid * _NS + sid
        pltpu.sync_copy(e2_hbm.at[0, pl.ds(wid * nchunk, nchunk)], src_v)
        pltpu.sync_copy(e2_hbm.at[1, pl.ds(wid * nchunk, nchunk)], dst_v)

        @pl.when(wid < nextra)
        def _():
            pltpu.sync_copy(e2_hbm.at[0, pl.ds(nchunk * _NW + wid, 1)], exs_v)
            pltpu.sync_copy(e2_hbm.at[1, pl.ds(nchunk * _NW + wid, 1)], exd_v)

        zeros = jnp.zeros((_LANES,), jnp.float32)

        def zero_body(i, _):
            for j in range(f // _LANES):
                rows0_v[i, pl.ds(j * _LANES, _LANES)] = zeros
            return 0

        lax.fori_loop(0, _CH, zero_body, 0)
        nfull, tail = rpt // _CH, rpt % _CH
        for k in range(nfull):
            pltpu.sync_copy(rows0_v,
                            acc_sh.at[pl.ds(sid * rpt + k * _CH, _CH)])
        if tail:
            pltpu.sync_copy(rows0_v.at[pl.ds(0, tail)],
                            acc_sh.at[pl.ds(sid * rpt + nfull * _CH, tail)])
        plsc.subcore_barrier()

        pltpu.async_copy(y_hbm.at[src_v.at[0]], rows0_v, sem0)

        def edge_body(jj, _):
            j0 = jj * 2
            pltpu.async_copy(y_hbm.at[src_v.at[j0 + 1]], rows1_v, sem1)
            pltpu.make_async_copy(y_hbm.at[src_v.at[j0]], rows0_v,
                                  sem0).wait()
            pltpu.sync_copy(rows0_v, acc_sh.at[dst_v.at[j0]], add=True)

            @pl.when(jj < nchunk // 2 - 1)
            def _():
                pltpu.async_copy(y_hbm.at[src_v.at[j0 + 2]], rows0_v, sem0)

            pltpu.make_async_copy(y_hbm.at[src_v.at[j0 + 1]], rows1_v,
                                  sem1).wait()
            pltpu.sync_copy(rows1_v, acc_sh.at[dst_v.at[j0 + 1]], add=True)
            return 0

        lax.fori_loop(0, nchunk // 2, edge_body, 0)

        @pl.when(wid < nextra)
        def _():
            pltpu.async_copy(y_hbm.at[exs_v.at[0]], rows0_v, sem0).wait()
            pltpu.sync_copy(rows0_v, acc_sh.at[exd_v.at[0]], add=True)

        plsc.subcore_barrier()
        pltpu.sync_copy(acc_sh.at[pl.ds(sid * rpt, rpt)],
                        out_hbm.at[pl.ds(sid * rpt, rpt), pl.ds(cid * f, f)])

    return agg_kernel(y, e2)


def _dis_from(degp_ref):
    deg = jnp.sum(degp_ref[0], axis=0) + 1.0
    return lax.rsqrt(deg)[:, None]


@functools.partial(jax.jit, static_argnames=("rb",))
def _tc_stage0(x, W1, ri2, W2b, *, rb):
    n, d = x.shape
    h = W1.shape[1]
    b = ri2.shape[0]
    nb = n // rb

    def body(x_ref, w1_ref, ri_ref, w2b_ref, xw_ref, r2_ref, rv_acc):
        i = pl.program_id(0)
        xb = x_ref[...]
        xw_ref[...] = jnp.dot(xb, w1_ref[...],
                              preferred_element_type=jnp.float32)
        gidx = i * rb + lax.broadcasted_iota(jnp.int32, (b, rb), 1)
        rsel = (ri_ref[...] == gidx).astype(jnp.float32)

        @pl.when(i == 0)
        def _():
            rv_acc[...] = jnp.zeros_like(rv_acc)

        rv_acc[...] += jnp.dot(rsel, xb, preferred_element_type=jnp.float32)

        @pl.when(i == nb - 1)
        def _():
            r2_ref[...] = jnp.dot(jnp.maximum(rv_acc[...], 0.0),
                                  w2b_ref[...],
                                  preferred_element_type=jnp.float32)

    return pl.pallas_call(
        body,
        grid=(nb,),
        in_specs=[
            pl.BlockSpec((rb, d), lambda i: (i, 0)),
            pl.BlockSpec((d, h), lambda i: (0, 0)),
            pl.BlockSpec((b, 1), lambda i: (0, 0)),
            pl.BlockSpec((d, h), lambda i: (0, 0)),
        ],
        out_specs=[
            pl.BlockSpec((rb, h), lambda i: (i, 0)),
            pl.BlockSpec((b, h), lambda i: (0, 0)),
        ],
        out_shape=[
            jax.ShapeDtypeStruct((n, h), jnp.float32),
            jax.ShapeDtypeStruct((b, h), jnp.float32),
        ],
        scratch_shapes=[pltpu.VMEM((b, d), jnp.float32)],
    )(x, W1, ri2, W2b)


@functools.partial(jax.jit, static_argnames=("rb",))
def _tc_stage1(degp, xw, *, rb):
    n, h = xw.shape
    nb = n // rb

    def body(degp_ref, xw_ref, y1_ref):
        dis = _dis_from(degp_ref)
        y1_ref[...] = dis * xw_ref[...]

    return pl.pallas_call(
        body,
        grid=(nb,),
        in_specs=[
            pl.BlockSpec((1, _NW, rb), lambda i: (i, 0, 0)),
            pl.BlockSpec((rb, h), lambda i: (i, 0)),
        ],
        out_specs=pl.BlockSpec((rb, h), lambda i: (i, 0)),
        out_shape=jax.ShapeDtypeStruct((n, h), jnp.float32),
    )(degp, xw)


@functools.partial(jax.jit, static_argnames=("rb",))
def _tc_stage2(S1, degp, y1, W2t, R2, bt3, b1r, *, rb):
    n = S1.shape[0]
    h = y1.shape[1]
    o = W2t.shape[1]
    b = R2.shape[0]
    nb = n // rb

    def body(s1_ref, degp_ref, y1_ref, w2t_ref, r2_ref, bt_ref, b1_ref,
             y2_ref):
        dis = _dis_from(degp_ref)
        sp = s1_ref[...]
        x2 = dis * (sp[:, :h] + sp[:, h:] + y1_ref[...]) + b1_ref[...]
        r = jnp.maximum(x2, 0.0)
        bt = bt_ref[0, 0, :]
        onehot = (bt[:, None]
                  == lax.broadcasted_iota(jnp.int32, (rb, b), 1)
                  ).astype(jnp.float32)
        hw2 = (jnp.dot(r, w2t_ref[...], preferred_element_type=jnp.float32)
               + jnp.dot(onehot, r2_ref[...],
                         preferred_element_type=jnp.float32))
        y2_ref[...] = dis * hw2

    return pl.pallas_call(
        body,
        grid=(nb,),
        in_specs=[
            pl.BlockSpec((rb, 128), lambda i: (i, 0)),
            pl.BlockSpec((1, _NW, rb), lambda i: (i, 0, 0)),
            pl.BlockSpec((rb, h), lambda i: (i, 0)),
            pl.BlockSpec((h, o), lambda i: (0, 0)),
            pl.BlockSpec((b, o), lambda i: (0, 0)),
            pl.BlockSpec((1, 1, rb), lambda i: (i, 0, 0)),
            pl.BlockSpec((1, h), lambda i: (0, 0)),
        ],
        out_specs=pl.BlockSpec((rb, o), lambda i: (i, 0)),
        out_shape=jax.ShapeDtypeStruct((n, o), jnp.float32),
    )(S1, degp, y1, W2t, R2, bt3, b1r)


@functools.partial(jax.jit, static_argnames=("rb",))
def _tc_stage3a(S1, degp, y1, b1r, bt3, ri2, *, rb):
    n = S1.shape[0]
    h = y1.shape[1]
    b = ri2.shape[0]
    nb = n // rb

    def body(s1_ref, degp_ref, y1_ref, b1_ref, bt_ref, ri_ref, x2r_ref,
             cnt_ref):
        i = pl.program_id(0)
        dis = _dis_from(degp_ref)
        sp = s1_ref[...]
        x2 = dis * (sp[:, :h] + sp[:, h:] + y1_ref[...]) + b1_ref[...]
        bt = bt_ref[0, 0, :]
        onehot_t = (lax.broadcasted_iota(jnp.int32, (b, rb), 0)
                    == bt[None, :]).astype(jnp.float32)
        gidx = i * rb + lax.broadcasted_iota(jnp.int32, (b, rb), 1)
        rsel = (ri_ref[...] == gidx).astype(jnp.float32)

        @pl.when(i == 0)
        def _():
            x2r_ref[...] = jnp.zeros_like(x2r_ref)
            cnt_ref[...] = jnp.zeros_like(cnt_ref)

        x2r_ref[...] += jnp.dot(rsel, x2,
                                preferred_element_type=jnp.float32)
        cnt_ref[...] += jnp.sum(onehot_t, axis=1, keepdims=True)

    return pl.pallas_call(
        body,
        grid=(nb,),
        in_specs=[
            pl.BlockSpec((rb, 128), lambda i: (i, 0)),
            pl.BlockSpec((1, _NW, rb), lambda i: (i, 0, 0)),
            pl.BlockSpec((rb, h), lambda i: (i, 0)),
            pl.BlockSpec((1, h), lambda i: (0, 0)),
            pl.BlockSpec((1, 1, rb), lambda i: (i, 0, 0)),
            pl.BlockSpec((b, 1), lambda i: (0, 0)),
        ],
        out_specs=[
            pl.BlockSpec((b, h), lambda i: (0, 0)),
            pl.BlockSpec((b, 1), lambda i: (0, 0)),
        ],
        out_shape=[
            jax.ShapeDtypeStruct((b, h), jnp.float32),
            jax.ShapeDtypeStruct((b, 1), jnp.float32),
        ],
    )(S1, degp, y1, b1r, bt3, ri2)


@functools.partial(jax.jit, static_argnames=("rb",))
def _tc_stage3b(S2, degp, y2, bt3, x2r, cnt, b2r, *, rb):
    n = S2.shape[0]
    o = y2.shape[1]
    b = x2r.shape[0]
    h = x2r.shape[1]
    nb = n // rb

    def body(s2_ref, degp_ref, y2_ref, bt_ref, x2r_ref, cnt_ref, b2_ref,
             out_ref, sums_acc):
        i = pl.program_id(0)
        dis = _dis_from(degp_ref)
        sp = s2_ref[...]
        g = jnp.maximum(
            dis * (sp[:, :o] + sp[:, o:] + y2_ref[...]) + b2_ref[...], 0.0)
        bt = bt_ref[0, 0, :]
        onehot_t = (lax.broadcasted_iota(jnp.int32, (b, rb), 0)
                    == bt[None, :]).astype(jnp.float32)

        @pl.when(i == 0)
        def _():
            sums_acc[...] = jnp.zeros_like(sums_acc)

        sums_acc[...] += jnp.dot(onehot_t, g,
                                 preferred_element_type=jnp.float32)

        @pl.when(i == nb - 1)
        def _():
            c = cnt_ref[...]
            mean = sums_acc[...] / jnp.maximum(c, 1.0)
            right = x2r_ref[...] * (c > 0.0).astype(jnp.float32)
            out_ref[...] = jnp.concatenate([mean, right], axis=1)

    return pl.pallas_call(
        body,
        grid=(nb,),
        in_specs=[
            pl.BlockSpec((rb, 128), lambda i: (i, 0)),
            pl.BlockSpec((1, _NW, rb), lambda i: (i, 0, 0)),
            pl.BlockSpec((rb, o), lambda i: (i, 0)),
            pl.BlockSpec((1, 1, rb), lambda i: (i, 0, 0)),
            pl.BlockSpec((b, h), lambda i: (0, 0)),
            pl.BlockSpec((b, 1), lambda i: (0, 0)),
            pl.BlockSpec((1, o), lambda i: (0, 0)),
        ],
        out_specs=pl.BlockSpec((b, o + h), lambda i: (0, 0)),
        out_shape=jax.ShapeDtypeStruct((b, o + h), jnp.float32),
        scratch_shapes=[pltpu.VMEM((b, o), jnp.float32)],
    )(S2, degp, y2, bt3, x2r, cnt, b2r)


def kernel(x, edge_index, root_index, batch, W1, b1, W2, b2):
    n, d = x.shape
    e = edge_index.shape[1]
    h = W1.shape[1]
    o = W2.shape[1]
    b = root_index.shape[0]
    rb = 2000

    e2 = edge_index.astype(jnp.int32).reshape(2, e // _CH, _CH)
    bt3 = batch.astype(jnp.int32).reshape(n // rb, 1, rb)
    ri2 = root_index.astype(jnp.int32).reshape(b, 1)
    b1r = b1.reshape(1, h)
    b2r = b2.reshape(1, o)
    W2t = W2[:h]
    W2b = W2[h:]

    degp = _degree_partials(e2, n=n, e=e, rb=rb)
    xw, R2 = _tc_stage0(x, W1, ri2, W2b, rb=rb)
    y1 = _tc_stage1(degp, xw, rb=rb)
    S1 = _edge_aggregate(y1, e2, n=n, f=h, e=e)
    y2 = _tc_stage2(S1, degp, y1, W2t, R2, bt3, b1r, rb=rb)
    S2 = _edge_aggregate(y2, e2, n=n, f=o, e=e)
    x2r, cnt = _tc_stage3a(S1, degp, y1, b1r, bt3, ri2, rb=rb)
    return _tc_stage3b(S2, degp, y2, bt3, x2r, cnt, b2r, rb=rb)


# speedup vs baseline: 1.0469x; 1.0004x over previous
import functools

import jax
import jax.numpy as jnp
from jax import lax
from jax.experimental import pallas as pl
from jax.experimental.pallas import tpu as pltpu
from jax.experimental.pallas import tpu_sc as plsc

_NC = 2
_NS = 16
_NW = _NC * _NS
_LANES = 16
_CH = 128


def _flat_worker_id():
    return lax.axis_index("c") * _NS + lax.axis_index("s")


@functools.partial(jax.jit, static_argnames=("n", "e", "rb"))
def _degree_partials(e2, *, n, e, rb):
    nchunk_tot = e // _CH
    nchunk = nchunk_tot // _NW
    nextra = nchunk_tot - nchunk * _NW
    nb = n // rb
    mesh = plsc.VectorSubcoreMesh(core_axis_name="c", subcore_axis_name="s")

    @functools.partial(
        pl.kernel,
        out_type=jax.ShapeDtypeStruct((nb, _NW, rb), jnp.float32),
        mesh=mesh,
        scratch_types=[
            pltpu.VMEM((nchunk, _CH), jnp.int32),
            pltpu.VMEM((1, _CH), jnp.int32),
            pltpu.VMEM((n,), jnp.float32),
        ],
        compiler_params=pltpu.CompilerParams(
            needs_layout_passes=False, use_tc_tiling_on_sc=False),
    )
    def deg_kernel(e2_hbm, out_hbm, idx_v, ex_v, acc_v):
        wid = _flat_worker_id()
        pltpu.sync_copy(e2_hbm.at[1, pl.ds(wid * nchunk, nchunk)], idx_v)

        @pl.when(wid < nextra)
        def _():
            pltpu.sync_copy(e2_hbm.at[1, pl.ds(nchunk * _NW + wid, 1)], ex_v)

        zeros = jnp.zeros((_LANES,), jnp.float32)

        def zero_body(i, _):
            acc_v[pl.ds(i * _LANES, _LANES)] = zeros
            return 0

        lax.fori_loop(0, n // _LANES, zero_body, 0)
        ones = jnp.ones((_LANES,), jnp.float32)

        def hist_body(i, _):
            for j in range(_CH // _LANES):
                idx = idx_v[i, pl.ds(j * _LANES, _LANES)]
                plsc.addupdate_scatter(acc_v, [idx], ones)
            return 0

        lax.fori_loop(0, nchunk, hist_body, 0)

        @pl.when(wid < nextra)
        def _():
            for j in range(_CH // _LANES):
                idx = ex_v[0, pl.ds(j * _LANES, _LANES)]
                plsc.addupdate_scatter(acc_v, [idx], ones)

        for blk in range(nb):
            pltpu.sync_copy(acc_v.at[pl.ds(blk * rb, rb)],
                            out_hbm.at[blk, wid])

    return deg_kernel(e2)


@functools.partial(jax.jit, static_argnames=("n", "f", "e"))
def _edge_aggregate(y, e2, *, n, f, e):
    nchunk_tot = e // _CH
    nchunk = nchunk_tot // _NW
    nextra = nchunk_tot - nchunk * _NW
    rpt = n // _NS
    mesh = plsc.VectorSubcoreMesh(core_axis_name="c", subcore_axis_name="s")

    @functools.partial(
        pl.kernel,
        out_type=jax.ShapeDtypeStruct((n, 128), jnp.float32),
        mesh=mesh,
        scratch_types=[
            pltpu.VMEM((nchunk, _CH), jnp.int32),
            pltpu.VMEM((nchunk, _CH), jnp.int32),
            pltpu.VMEM((1, _CH), jnp.int32),
            pltpu.VMEM((1, _CH), jnp.int32),
            pltpu.VMEM((_CH, f), jnp.float32),
            pltpu.VMEM((_CH, f), jnp.float32),
            pltpu.VMEM_SHARED((n, f), jnp.float32),
            pltpu.SemaphoreType.DMA,
            pltpu.SemaphoreType.DMA,
        ],
        compiler_params=pltpu.CompilerParams(use_tc_tiling_on_sc=False),
    )
    def agg_kernel(y_hbm, e2_hbm, out_hbm, src_v, dst_v, exs_v, exd_v,
                   rows0_v, rows1_v, acc_sh, sem0, sem1):
        cid = lax.axis_index("c")
        sid = lax.axis_index("s")
        wid = cid * _NS + sid
        pltpu.sync_copy(e2_hbm.at[0, pl.ds(wid * nchunk, nchunk)], src_v)
        pltpu.sync_copy(e2_hbm.at[1, pl.ds(wid * nchunk, nchunk)], dst_v)

        @pl.when(wid < nextra)
        def _():
            pltpu.sync_copy(e2_hbm.at[0, pl.ds(nchunk * _NW + wid, 1)], exs_v)
            pltpu.sync_copy(e2_hbm.at[1, pl.ds(nchunk * _NW + wid, 1)], exd_v)

        zeros = jnp.zeros((_LANES,), jnp.float32)

        def zero_body(i, _):
            for j in range(f // _LANES):
                rows0_v[i, pl.ds(j * _LANES, _LANES)] = zeros
            return 0

        lax.fori_loop(0, _CH, zero_body, 0)
        nfull, tail = rpt // _CH, rpt % _CH
        for k in range(nfull):
            pltpu.sync_copy(rows0_v,
                            acc_sh.at[pl.ds(sid * rpt + k * _CH, _CH)])
        if tail:
            pltpu.sync_copy(rows0_v.at[pl.ds(0, tail)],
                            acc_sh.at[pl.ds(sid * rpt + nfull * _CH, tail)])
        plsc.subcore_barrier()

        pltpu.async_copy(y_hbm.at[src_v.at[0]], rows0_v, sem0)

        def edge_body(jj, _):
            j0 = jj * 2
            pltpu.async_copy(y_hbm.at[src_v.at[j0 + 1]], rows1_v, sem1)
            pltpu.make_async_copy(y_hbm.at[src_v.at[j0]], rows0_v,
                                  sem0).wait()
            pltpu.sync_copy(rows0_v, acc_sh.at[dst_v.at[j0]], add=True)

            @pl.when(jj < nchunk // 2 - 1)
            def _():
                pltpu.async_copy(y_hbm.at[src_v.at[j0 + 2]], rows0_v, sem0)

            pltpu.make_async_copy(y_hbm.at[src_v.at[j0 + 1]], rows1_v,
                                  sem1).wait()
            pltpu.sync_copy(rows1_v, acc_sh.at[dst_v.at[j0 + 1]], add=True)
            return 0

        lax.fori_loop(0, nchunk // 2, edge_body, 0)

        @pl.when(wid < nextra)
        def _():
            pltpu.async_copy(y_hbm.at[exs_v.at[0]], rows0_v, sem0).wait()
            pltpu.sync_copy(rows0_v, acc_sh.at[exd_v.at[0]], add=True)

        plsc.subcore_barrier()
        pltpu.sync_copy(acc_sh.at[pl.ds(sid * rpt, rpt)],
                        out_hbm.at[pl.ds(sid * rpt, rpt), pl.ds(cid * f, f)])

    return agg_kernel(y, e2)


def _dis_from(degp_ref):
    deg = jnp.sum(degp_ref[0], axis=0) + 1.0
    return lax.rsqrt(deg)[:, None]


@functools.partial(jax.jit, static_argnames=("rb",))
def _tc_stage0(x, W1, ri2, W2b, *, rb):
    n, d = x.shape
    h = W1.shape[1]
    b = ri2.shape[0]
    nb = n // rb

    def body(x_ref, w1_ref, ri_ref, w2b_ref, xw_ref, r2_ref, rv_acc):
        i = pl.program_id(0)
        xb = x_ref[...]
        xw_ref[...] = jnp.dot(xb, w1_ref[...],
                              preferred_element_type=jnp.float32)
        gidx = i * rb + lax.broadcasted_iota(jnp.int32, (b, rb), 1)
        rsel = (ri_ref[...] == gidx).astype(jnp.float32)

        @pl.when(i == 0)
        def _():
            rv_acc[...] = jnp.zeros_like(rv_acc)

        rv_acc[...] += jnp.dot(rsel, xb, preferred_element_type=jnp.float32)

        @pl.when(i == nb - 1)
        def _():
            r2_ref[...] = jnp.dot(jnp.maximum(rv_acc[...], 0.0),
                                  w2b_ref[...],
                                  preferred_element_type=jnp.float32)

    return pl.pallas_call(
        body,
        grid=(nb,),
        in_specs=[
            pl.BlockSpec((rb, d), lambda i: (i, 0)),
            pl.BlockSpec((d, h), lambda i: (0, 0)),
            pl.BlockSpec((b, 1), lambda i: (0, 0)),
            pl.BlockSpec((d, h), lambda i: (0, 0)),
        ],
        out_specs=[
            pl.BlockSpec((rb, h), lambda i: (i, 0)),
            pl.BlockSpec((b, h), lambda i: (0, 0)),
        ],
        out_shape=[
            jax.ShapeDtypeStruct((n, h), jnp.float32),
            jax.ShapeDtypeStruct((b, h), jnp.float32),
        ],
        scratch_shapes=[pltpu.VMEM((b, d), jnp.float32)],
    )(x, W1, ri2, W2b)


@functools.partial(jax.jit, static_argnames=("rb",))
def _tc_stage1(degp, xw, *, rb):
    n, h = xw.shape
    nb = n // rb

    def body(degp_ref, xw_ref, y1_ref):
        dis = _dis_from(degp_ref)
        y1_ref[...] = dis * xw_ref[...]

    return pl.pallas_call(
        body,
        grid=(nb,),
        in_specs=[
            pl.BlockSpec((1, _NW, rb), lambda i: (i, 0, 0)),
            pl.BlockSpec((rb, h), lambda i: (i, 0)),
        ],
        out_specs=pl.BlockSpec((rb, h), lambda i: (i, 0)),
        out_shape=jax.ShapeDtypeStruct((n, h), jnp.float32),
    )(degp, xw)


@functools.partial(jax.jit, static_argnames=("rb", "h"))
def _tc_stage2(S1, degp, y1f, W2t, R2, bt3, b1r, *, rb, h):
    n = S1.shape[0]
    o = W2t.shape[1]
    b = R2.shape[0]
    nb = n // rb

    def body(s1_ref, degp_ref, y1_ref, w2t_ref, r2_ref, bt_ref, b1_ref,
             y2_ref):
        dis = _dis_from(degp_ref)
        sp = s1_ref[...]
        y1 = y1_ref[...]
        x2 = dis * (sp[:, :h] + sp[:, h:] + y1) + b1_ref[...]
        r = jnp.maximum(x2, 0.0)
        bt = bt_ref[0, 0, :]
        onehot = (bt[:, None]
                  == lax.broadcasted_iota(jnp.int32, (rb, b), 1)
                  ).astype(jnp.float32)
        hw2 = (jnp.dot(r, w2t_ref[...], preferred_element_type=jnp.float32)
               + jnp.dot(onehot, r2_ref[...],
                         preferred_element_type=jnp.float32))
        y2_ref[...] = dis * hw2

    return pl.pallas_call(
        body,
        grid=(nb,),
        in_specs=[
            pl.BlockSpec((rb, 128), lambda i: (i, 0)),
            pl.BlockSpec((1, _NW, rb), lambda i: (i, 0, 0)),
            pl.BlockSpec((rb, h), lambda i: (i, 0)),
            pl.BlockSpec((h, o), lambda i: (0, 0)),
            pl.BlockSpec((b, o), lambda i: (0, 0)),
            pl.BlockSpec((1, 1, rb), lambda i: (i, 0, 0)),
            pl.BlockSpec((1, h), lambda i: (0, 0)),
        ],
        out_specs=pl.BlockSpec((rb, o), lambda i: (i, 0)),
        out_shape=jax.ShapeDtypeStruct((n, o), jnp.float32),
    )(S1, degp, y1f, W2t, R2, bt3, b1r)


@functools.partial(jax.jit, static_argnames=("rb", "h"))
def _tc_stage3a(S1, degp, y1f, b1r, bt3, ri2, *, rb, h):
    n = S1.shape[0]
    b = ri2.shape[0]
    nb = n // rb

    def body(s1_ref, degp_ref, y1_ref, b1_ref, bt_ref, ri_ref, x2r_ref,
             cnt_ref):
        i = pl.program_id(0)
        dis = _dis_from(degp_ref)
        sp = s1_ref[...]
        y1 = y1_ref[...]
        x2 = dis * (sp[:, :h] + sp[:, h:] + y1) + b1_ref[...]
        bt = bt_ref[0, 0, :]
        onehot_t = (lax.broadcasted_iota(jnp.int32, (b, rb), 0)
                    == bt[None, :]).astype(jnp.float32)
        gidx = i * rb + lax.broadcasted_iota(jnp.int32, (b, rb), 1)
        rsel = (ri_ref[...] == gidx).astype(jnp.float32)

        @pl.when(i == 0)
        def _():
            x2r_ref[...] = jnp.zeros_like(x2r_ref)
            cnt_ref[...] = jnp.zeros_like(cnt_ref)

        x2r_ref[...] += jnp.dot(rsel, x2,
                                preferred_element_type=jnp.float32)
        cnt_ref[...] += jnp.sum(onehot_t, axis=1, keepdims=True)

    return pl.pallas_call(
        body,
        grid=(nb,),
        in_specs=[
            pl.BlockSpec((rb, 128), lambda i: (i, 0)),
            pl.BlockSpec((1, _NW, rb), lambda i: (i, 0, 0)),
            pl.BlockSpec((rb, h), lambda i: (i, 0)),
            pl.BlockSpec((1, h), lambda i: (0, 0)),
            pl.BlockSpec((1, 1, rb), lambda i: (i, 0, 0)),
            pl.BlockSpec((b, 1), lambda i: (0, 0)),
        ],
        out_specs=[
            pl.BlockSpec((b, h), lambda i: (0, 0)),
            pl.BlockSpec((b, 1), lambda i: (0, 0)),
        ],
        out_shape=[
            jax.ShapeDtypeStruct((b, h), jnp.float32),
            jax.ShapeDtypeStruct((b, 1), jnp.float32),
        ],
    )(S1, degp, y1f, b1r, bt3, ri2)


@functools.partial(jax.jit, static_argnames=("rb", "o"))
def _tc_stage3b(S2, degp, y2f, bt3, x2r, cnt, b2r, *, rb, o):
    n = S2.shape[0]
    b = x2r.shape[0]
    h = x2r.shape[1]
    nb = n // rb

    def body(s2_ref, degp_ref, y2_ref, bt_ref, x2r_ref, cnt_ref, b2_ref,
             out_ref, sums_acc):
        i = pl.program_id(0)
        dis = _dis_from(degp_ref)
        sp = s2_ref[...]
        y2 = y2_ref[...]
        g = jnp.maximum(
            dis * (sp[:, :o] + sp[:, o:] + y2) + b2_ref[...], 0.0)
        bt = bt_ref[0, 0, :]
        onehot_t = (lax.broadcasted_iota(jnp.int32, (b, rb), 0)
                    == bt[None, :]).astype(jnp.float32)

        @pl.when(i == 0)
        def _():
            sums_acc[...] = jnp.zeros_like(sums_acc)

        sums_acc[...] += jnp.dot(onehot_t, g,
                                 preferred_element_type=jnp.float32)

        @pl.when(i == nb - 1)
        def _():
            c = cnt_ref[...]
            mean = sums_acc[...] / jnp.maximum(c, 1.0)
            right = x2r_ref[...] * (c > 0.0).astype(jnp.float32)
            out_ref[...] = jnp.concatenate([mean, right], axis=1)

    return pl.pallas_call(
        body,
        grid=(nb,),
        in_specs=[
            pl.BlockSpec((rb, 128), lambda i: (i, 0)),
            pl.BlockSpec((1, _NW, rb), lambda i: (i, 0, 0)),
            pl.BlockSpec((rb, o), lambda i: (i, 0)),
            pl.BlockSpec((1, 1, rb), lambda i: (i, 0, 0)),
            pl.BlockSpec((b, h), lambda i: (0, 0)),
            pl.BlockSpec((b, 1), lambda i: (0, 0)),
            pl.BlockSpec((1, o), lambda i: (0, 0)),
        ],
        out_specs=pl.BlockSpec((b, o + h), lambda i: (0, 0)),
        out_shape=jax.ShapeDtypeStruct((b, o + h), jnp.float32),
        scratch_shapes=[pltpu.VMEM((b, o), jnp.float32)],
    )(S2, degp, y2f, bt3, x2r, cnt, b2r)


def kernel(x, edge_index, root_index, batch, W1, b1, W2, b2):
    n, d = x.shape
    e = edge_index.shape[1]
    h = W1.shape[1]
    o = W2.shape[1]
    b = root_index.shape[0]
    rb = 2000

    e2 = edge_index.astype(jnp.int32).reshape(2, e // _CH, _CH)
    bt3 = batch.astype(jnp.int32).reshape(n // rb, 1, rb)
    ri2 = root_index.astype(jnp.int32).reshape(b, 1)
    b1r = b1.reshape(1, h)
    b2r = b2.reshape(1, o)
    W2t = W2[:h]
    W2b = W2[h:]

    degp = _degree_partials(e2, n=n, e=e, rb=rb)
    xw, R2 = _tc_stage0(x, W1, ri2, W2b, rb=rb)
    y1 = _tc_stage1(degp, xw, rb=rb)
    S1 = _edge_aggregate(y1, e2, n=n, f=h, e=e)
    y2 = _tc_stage2(S1, degp, y1, W2t, R2, bt3, b1r, rb=rb, h=h)
    S2 = _edge_aggregate(y2, e2, n=n, f=o, e=e)
    x2r, cnt = _tc_stage3a(S1, degp, y1, b1r, bt3, ri2, rb=rb, h=h)
    return _tc_stage3b(S2, degp, y2, bt3, x2r, cnt, b2r, rb=rb, o=o)
